# Initial kernel scaffold; baseline (speedup 1.0000x reference)
import jax
import jax.numpy as jnp
from jax.experimental import pallas as pl


def kernel(pcs):
    raise NotImplementedError("write your pallas kernel here")


# speedup vs baseline: 20.4254x; 20.4254x over previous
import jax
import jax.numpy as jnp
from jax.experimental import pallas as pl

_K = 10
_NSEEDS = 20
_INV_H2 = 10000.0


def _knn_repulsion_body(xyz_ref, out_ref):
    x = xyz_ref[0]
    y = xyz_ref[1]
    z = xyz_ref[2]
    b, n = x.shape
    lane = jax.lax.broadcasted_iota(jnp.int32, (b, n), 1)

    def gather_coords(far):
        m = lane == far
        zero = jnp.zeros_like(x)
        cx = jnp.sum(jnp.where(m, x, zero), axis=1, keepdims=True)
        cy = jnp.sum(jnp.where(m, y, zero), axis=1, keepdims=True)
        cz = jnp.sum(jnp.where(m, z, zero), axis=1, keepdims=True)
        return cx, cy, cz

    def sqdist(c):
        cx, cy, cz = c
        dx = x - cx
        dy = y - cy
        dz = z - cz
        return dx * dx + dy * dy + dz * dz

    far = jnp.zeros((b, 1), jnp.int32)
    seed_coords = []
    distance = jnp.full((b, n), 1e10, jnp.float32)
    for s in range(_NSEEDS):
        c = gather_coords(far)
        seed_coords.append(c)
        if s == _NSEEDS - 1:
            break
        distance = jnp.minimum(distance, sqdist(c))
        mx = jnp.max(distance, axis=1, keepdims=True)
        far = jnp.min(jnp.where(distance == mx, lane, n), axis=1, keepdims=True)

    def pop_min(d):
        m = jnp.min(d, axis=1, keepdims=True)
        pos = jnp.min(jnp.where(d == m, lane, n), axis=1, keepdims=True)
        return m, jnp.where(lane == pos, jnp.float32(jnp.inf), d)

    total = jnp.zeros((b, 1), jnp.float32)
    for s in range(_NSEEDS):
        d = jnp.sqrt(sqdist(seed_coords[s]) + 1e-12)
        _, d = pop_min(d)

        def kbody(_, carry):
            d, acc = carry
            m, d = pop_min(d)
            acc = acc + (-m) * jnp.exp(-(m * m) * _INV_H2)
            return d, acc

        d, total = jax.lax.fori_loop(0, _K, kbody, (d, total))
    out_ref[...] = jnp.sum(total, axis=0, keepdims=True) * (1.0 / b)


def kernel(pcs):
    xyz = jnp.transpose(pcs, (2, 0, 1))
    out = pl.pallas_call(
        _knn_repulsion_body,
        out_shape=jax.ShapeDtypeStruct((1, 1), jnp.float32),
    )(xyz)
    return out[0, 0]


# speedup vs baseline: 53.1788x; 2.6036x over previous
import jax
import jax.numpy as jnp
from jax.experimental import pallas as pl

_K = 10
_NSEEDS = 20
_INV_H2 = 10000.0


def _knn_repulsion_body(xyz_ref, out_ref):
    x = xyz_ref[0]
    y = xyz_ref[1]
    z = xyz_ref[2]
    b, n = x.shape
    lanef = jax.lax.broadcasted_iota(jnp.int32, (b, n), 1).astype(jnp.float32)
    nf = jnp.float32(n)
    inf = jnp.float32(jnp.inf)
    zero = jnp.zeros_like(x)

    def gather_coords(farf):
        m = lanef == farf
        cx = jnp.sum(jnp.where(m, x, zero), axis=1, keepdims=True)
        cy = jnp.sum(jnp.where(m, y, zero), axis=1, keepdims=True)
        cz = jnp.sum(jnp.where(m, z, zero), axis=1, keepdims=True)
        return cx, cy, cz

    def sqdist(c):
        cx, cy, cz = c
        dx = x - cx
        dy = y - cy
        dz = z - cz
        return dx * dx + dy * dy + dz * dz

    farf = jnp.zeros((b, 1), jnp.float32)
    seeds = []
    distance = jnp.full((b, n), 1e10, jnp.float32)
    for s in range(_NSEEDS):
        c = gather_coords(farf)
        seeds.append((farf, c))
        if s == _NSEEDS - 1:
            break
        distance = jnp.minimum(distance, sqdist(c))
        mx = jnp.max(distance, axis=1, keepdims=True)
        farf = jnp.min(jnp.where(distance == mx, lanef, nf), axis=1, keepdims=True)

    total = jnp.zeros((b, 1), jnp.float32)
    for s in range(_NSEEDS):
        sfarf, c = seeds[s]
        q = jnp.where(lanef == sfarf, inf, sqdist(c))
        acc = jnp.zeros((b, 1), jnp.float32)
        for _ in range(_K):
            m = jnp.min(q, axis=1, keepdims=True)
            posf = jnp.min(jnp.where(q == m, lanef, nf), axis=1, keepdims=True)
            q = jnp.where(lanef == posf, inf, q)
            t = jnp.sqrt(m + 1e-12)
            acc = acc + (-t) * jnp.exp(-(t * t) * _INV_H2)
        total = total + acc
    out_ref[...] = jnp.sum(total, axis=0, keepdims=True) * (1.0 / b)


def kernel(pcs):
    xyz = jnp.transpose(pcs, (2, 0, 1))
    out = pl.pallas_call(
        _knn_repulsion_body,
        out_shape=jax.ShapeDtypeStruct((1, 1), jnp.float32),
    )(xyz)
    return out[0, 0]
